# baseline (speedup 1.0000x reference)
import functools

import jax
import jax.numpy as jnp
from jax import lax
from jax.experimental import pallas as pl
from jax.experimental.pallas import tpu as pltpu
from jax.experimental.pallas import tpu_sc as plsc

DIM = 128
BATCH = 16384
CHUNK = 128


@functools.lru_cache(maxsize=None)
def _make_gather_kernel(V, D, B):
    info = plsc.get_sparse_core_info()
    NC, NS = info.num_cores, info.num_subcores
    NW = NC * NS
    assert B % (8 * NW) == 0
    b_per_w = B // NW
    n_chunks = max(1, b_per_w // CHUNK)
    chunk = b_per_w // n_chunks
    mesh = plsc.VectorSubcoreMesh(core_axis_name="c", subcore_axis_name="s")

    @functools.partial(
        pl.kernel,
        mesh=mesh,
        out_type=jax.ShapeDtypeStruct((B, D), jnp.float32),
        scratch_types=[
            pltpu.VMEM((n_chunks, chunk), jnp.int32),
            pltpu.VMEM((n_chunks, chunk, D), jnp.float32),
            pltpu.SemaphoreType.DMA,
            pltpu.SemaphoreType.DMA,
        ],
    )
    def gather_kernel(table_hbm, idx_hbm, out_hbm, idx_v, rows_v, sem_g, sem_s):
        wid = lax.axis_index("s") * NC + lax.axis_index("c")
        base = wid * b_per_w
        pltpu.sync_copy(idx_hbm.at[wid], idx_v)
        gathers = [
            pltpu.async_copy(table_hbm.at[idx_v.at[c]], rows_v.at[c], sem_g)
            for c in range(n_chunks)
        ]
        scatters = []
        for c in range(n_chunks):
            gathers[c].wait()
            scatters.append(
                pltpu.async_copy(
                    rows_v.at[c], out_hbm.at[pl.ds(base + c * chunk, chunk)], sem_s
                )
            )
        for s in scatters:
            s.wait()

    def call(cache, timesteps):
        idx = timesteps.reshape(NW, n_chunks, chunk)
        return gather_kernel(cache, idx)

    return call


def kernel(timesteps, cache):
    k = _make_gather_kernel(cache.shape[0], cache.shape[1], timesteps.shape[0])
    return k(cache, timesteps.astype(jnp.int32))
